# Initial kernel scaffold; baseline (speedup 1.0000x reference)
import jax
import jax.numpy as jnp
from jax.experimental import pallas as pl


def kernel(hidden_states, lmask, input_ids, attention_mask, W_label, b_label):
    raise NotImplementedError("write your pallas kernel here")


# speedup vs baseline: 4.9462x; 4.9462x over previous
import functools

import jax
import jax.numpy as jnp
from jax.experimental import pallas as pl
from jax.experimental.pallas import tpu as pltpu

B, L, H = 16, 4096, 768
MAX_LABEL = 24
ROWS = 32
CH = 512
NCH = L // CH


def _seg_body(lm_ref, hs_ref, sums_ref, counts_ref):
    c = pl.program_id(1)
    labels = lm_ref[0, 0]
    rows = jax.lax.broadcasted_iota(jnp.int32, (ROWS, CH), 0)
    oh = (labels == rows).astype(jnp.float32)
    contrib = jax.lax.dot(oh, hs_ref[0],
                          precision=jax.lax.Precision.HIGHEST,
                          preferred_element_type=jnp.float32)
    cnt = jnp.broadcast_to(jnp.sum(oh, axis=1)[:, None], (ROWS, 128))

    @pl.when(c == 0)
    def _():
        sums_ref[0] = contrib
        counts_ref[0] = cnt

    @pl.when(c != 0)
    def _():
        sums_ref[0] += contrib
        counts_ref[0] += cnt


def _fin_body(sums_ref, counts_ref, w_ref, b_ref, out_ref, valid_ref):
    cnt = counts_ref[:, 0:1]
    valid = (cnt > 0).astype(jnp.float32)
    mean = sums_ref[...] / jnp.maximum(cnt, 1.0)
    proj = jax.lax.dot(mean, w_ref[...],
                       precision=jax.lax.Precision.HIGHEST,
                       preferred_element_type=jnp.float32) + b_ref[...]
    out_ref[...] = proj * valid
    valid_ref[...] = jnp.broadcast_to(valid, (B * ROWS, 128))


@functools.partial(jax.jit, static_argnames=())
def _segment_sums(hidden_states, lmask):
    lm3 = lmask.astype(jnp.int32).reshape(B, NCH, 1, CH)
    sums, counts = pl.pallas_call(
        _seg_body,
        grid=(B, NCH),
        in_specs=[
            pl.BlockSpec((1, 1, 1, CH), lambda b, c: (b, c, 0, 0)),
            pl.BlockSpec((1, CH, H), lambda b, c: (b, c, 0)),
        ],
        out_specs=[
            pl.BlockSpec((1, ROWS, H), lambda b, c: (b, 0, 0)),
            pl.BlockSpec((1, ROWS, 128), lambda b, c: (b, 0, 0)),
        ],
        out_shape=[
            jax.ShapeDtypeStruct((B, ROWS, H), jnp.float32),
            jax.ShapeDtypeStruct((B, ROWS, 128), jnp.float32),
        ],
    )(lm3, hidden_states)
    return sums, counts


def _finish(sums, counts, W_label, b_label):
    out, valid = pl.pallas_call(
        _fin_body,
        out_shape=[
            jax.ShapeDtypeStruct((B * ROWS, H), jnp.float32),
            jax.ShapeDtypeStruct((B * ROWS, 128), jnp.float32),
        ],
    )(sums.reshape(B * ROWS, H), counts.reshape(B * ROWS, 128),
      W_label, b_label.reshape(1, H))
    return out, valid


def kernel(hidden_states, lmask, input_ids, attention_mask, W_label, b_label):
    sums, counts = _segment_sums(hidden_states, lmask)
    out, valid = _finish(sums, counts, W_label, b_label)
    out3 = out.reshape(B, ROWS, H)[:, 1:MAX_LABEL + 1, :]
    aggregated = out3.reshape(B * MAX_LABEL, H)
    valid_mask = (valid.reshape(B, ROWS, 128)[:, 1:MAX_LABEL + 1, 0] > 0
                  ).reshape(B * MAX_LABEL)
    all_batch_ids = jnp.repeat(jnp.arange(B), MAX_LABEL)
    all_label_ids = jnp.tile(jnp.arange(1, MAX_LABEL + 1), B)
    return aggregated, all_batch_ids, all_label_ids, valid_mask


# speedup vs baseline: 6.5075x; 1.3157x over previous
import functools

import jax
import jax.numpy as jnp
from jax.experimental import pallas as pl
from jax.experimental.pallas import tpu as pltpu

B, L, H = 16, 4096, 768
MAX_LABEL = 24
ROWS = 32
CH = 512
NCH = L // CH


def _seg_body(lm_ref, hs_ref, sums_ref, counts_ref):
    c = pl.program_id(1)
    labels = lm_ref[0, 0]
    rows = jax.lax.broadcasted_iota(jnp.int32, (ROWS, CH), 0)
    oh = (labels == rows).astype(jnp.float32)
    contrib = jax.lax.dot(oh, hs_ref[0],
                          precision=jax.lax.Precision.DEFAULT,
                          preferred_element_type=jnp.float32)
    cnt = jnp.broadcast_to(jnp.sum(oh, axis=1)[:, None], (ROWS, 128))

    @pl.when(c == 0)
    def _():
        sums_ref[0] = contrib
        counts_ref[0] = cnt

    @pl.when(c != 0)
    def _():
        sums_ref[0] += contrib
        counts_ref[0] += cnt


def _fin_body(sums_ref, counts_ref, w_ref, b_ref, out_ref, valid_ref):
    cnt = counts_ref[:, 0:1]
    valid = (cnt > 0).astype(jnp.float32)
    mean = sums_ref[...] / jnp.maximum(cnt, 1.0)
    proj = jax.lax.dot(mean, w_ref[...],
                       precision=jax.lax.Precision.HIGHEST,
                       preferred_element_type=jnp.float32) + b_ref[...]
    out_ref[...] = proj * valid
    valid_ref[...] = jnp.broadcast_to(valid, (B * ROWS, 128))


@functools.partial(jax.jit, static_argnames=())
def _segment_sums(hidden_states, lmask):
    lm3 = lmask.astype(jnp.int32).reshape(B, NCH, 1, CH)
    sums, counts = pl.pallas_call(
        _seg_body,
        grid=(B, NCH),
        in_specs=[
            pl.BlockSpec((1, 1, 1, CH), lambda b, c: (b, c, 0, 0)),
            pl.BlockSpec((1, CH, H), lambda b, c: (b, c, 0)),
        ],
        out_specs=[
            pl.BlockSpec((1, ROWS, H), lambda b, c: (b, 0, 0)),
            pl.BlockSpec((1, ROWS, 128), lambda b, c: (b, 0, 0)),
        ],
        out_shape=[
            jax.ShapeDtypeStruct((B, ROWS, H), jnp.float32),
            jax.ShapeDtypeStruct((B, ROWS, 128), jnp.float32),
        ],
    )(lm3, hidden_states)
    return sums, counts


def _finish(sums, counts, W_label, b_label):
    out, valid = pl.pallas_call(
        _fin_body,
        out_shape=[
            jax.ShapeDtypeStruct((B * ROWS, H), jnp.float32),
            jax.ShapeDtypeStruct((B * ROWS, 128), jnp.float32),
        ],
    )(sums.reshape(B * ROWS, H), counts.reshape(B * ROWS, 128),
      W_label, b_label.reshape(1, H))
    return out, valid


def kernel(hidden_states, lmask, input_ids, attention_mask, W_label, b_label):
    sums, counts = _segment_sums(hidden_states, lmask)
    out, valid = _finish(sums, counts, W_label, b_label)
    out3 = out.reshape(B, ROWS, H)[:, 1:MAX_LABEL + 1, :]
    aggregated = out3.reshape(B * MAX_LABEL, H)
    valid_mask = (valid.reshape(B, ROWS, 128)[:, 1:MAX_LABEL + 1, 0] > 0
                  ).reshape(B * MAX_LABEL)
    all_batch_ids = jnp.repeat(jnp.arange(B), MAX_LABEL)
    all_label_ids = jnp.tile(jnp.arange(1, MAX_LABEL + 1), B)
    return aggregated, all_batch_ids, all_label_ids, valid_mask


# speedup vs baseline: 9.3875x; 1.4426x over previous
import functools

import jax
import jax.numpy as jnp
from jax.experimental import pallas as pl
from jax.experimental.pallas import tpu as pltpu

B, L, H = 16, 4096, 768
MAX_LABEL = 24
ROWS = 32
CH = 1024
NCH = L // CH


def _seg_body(lm_ref, hs_ref, sums_ref, counts_ref):
    c = pl.program_id(1)
    labels = lm_ref[0, 0]
    rows = jax.lax.broadcasted_iota(jnp.int32, (ROWS, CH), 0)
    oh = (labels == rows).astype(jnp.float32)
    contrib = jax.lax.dot(oh, hs_ref[0],
                          precision=jax.lax.Precision.DEFAULT,
                          preferred_element_type=jnp.float32)
    cnt = jnp.broadcast_to(jnp.sum(oh, axis=1)[:, None], (ROWS, 128))

    @pl.when(c == 0)
    def _():
        sums_ref[0] = contrib
        counts_ref[0] = cnt

    @pl.when(c != 0)
    def _():
        sums_ref[0] += contrib
        counts_ref[0] += cnt


def _fin_body(sums_ref, counts_ref, w_ref, b_ref, out_ref, valid_ref):
    cnt = counts_ref[:, 0:1]
    valid = (cnt > 0).astype(jnp.float32)
    mean = sums_ref[...] / jnp.maximum(cnt, 1.0)
    proj = jax.lax.dot(mean, w_ref[...],
                       precision=jax.lax.Precision.DEFAULT,
                       preferred_element_type=jnp.float32) + b_ref[...]
    out_ref[...] = proj * valid
    valid_ref[...] = jnp.broadcast_to(valid, (B * ROWS, 128))


@functools.partial(jax.jit, static_argnames=())
def _segment_sums(hidden_states, lmask):
    lm3 = lmask.astype(jnp.int32).reshape(B, NCH, 1, CH)
    sums, counts = pl.pallas_call(
        _seg_body,
        grid=(B, NCH),
        in_specs=[
            pl.BlockSpec((1, 1, 1, CH), lambda b, c: (b, c, 0, 0)),
            pl.BlockSpec((1, CH, H), lambda b, c: (b, c, 0)),
        ],
        out_specs=[
            pl.BlockSpec((1, ROWS, H), lambda b, c: (b, 0, 0)),
            pl.BlockSpec((1, ROWS, 128), lambda b, c: (b, 0, 0)),
        ],
        out_shape=[
            jax.ShapeDtypeStruct((B, ROWS, H), jnp.float32),
            jax.ShapeDtypeStruct((B, ROWS, 128), jnp.float32),
        ],
    )(lm3, hidden_states)
    return sums, counts


def _finish(sums, counts, W_label, b_label):
    out, valid = pl.pallas_call(
        _fin_body,
        out_shape=[
            jax.ShapeDtypeStruct((B * ROWS, H), jnp.float32),
            jax.ShapeDtypeStruct((B * ROWS, 128), jnp.float32),
        ],
    )(sums.reshape(B * ROWS, H), counts.reshape(B * ROWS, 128),
      W_label, b_label.reshape(1, H))
    return out, valid


def kernel(hidden_states, lmask, input_ids, attention_mask, W_label, b_label):
    sums, counts = _segment_sums(hidden_states, lmask)
    out, valid = _finish(sums, counts, W_label, b_label)
    out3 = out.reshape(B, ROWS, H)[:, 1:MAX_LABEL + 1, :]
    aggregated = out3.reshape(B * MAX_LABEL, H)
    valid_mask = (valid.reshape(B, ROWS, 128)[:, 1:MAX_LABEL + 1, 0] > 0
                  ).reshape(B * MAX_LABEL)
    all_batch_ids = jnp.repeat(jnp.arange(B), MAX_LABEL)
    all_label_ids = jnp.tile(jnp.arange(1, MAX_LABEL + 1), B)
    return aggregated, all_batch_ids, all_label_ids, valid_mask


# speedup vs baseline: 11.6807x; 1.2443x over previous
import functools

import jax
import jax.numpy as jnp
from jax.experimental import pallas as pl
from jax.experimental.pallas import tpu as pltpu

B, L, H = 16, 4096, 768
MAX_LABEL = 24
ROWS = 32
CH = 2048
NCH = L // CH


def _seg_body(lm_ref, hs_ref, sums_ref, counts_ref):
    c = pl.program_id(1)
    labels = lm_ref[0, 0]
    rows = jax.lax.broadcasted_iota(jnp.int32, (ROWS, CH), 0)
    oh = (labels == rows).astype(jnp.float32)
    contrib = jax.lax.dot(oh, hs_ref[0],
                          precision=jax.lax.Precision.DEFAULT,
                          preferred_element_type=jnp.float32)
    cnt = jnp.broadcast_to(jnp.sum(oh, axis=1)[:, None], (ROWS, 128))

    @pl.when(c == 0)
    def _():
        sums_ref[0] = contrib
        counts_ref[0] = cnt

    @pl.when(c != 0)
    def _():
        sums_ref[0] += contrib
        counts_ref[0] += cnt


def _fin_body(sums_ref, counts_ref, w_ref, b_ref, out_ref, valid_ref):
    cnt = counts_ref[:, 0:1]
    valid = (cnt > 0).astype(jnp.float32)
    mean = sums_ref[...] / jnp.maximum(cnt, 1.0)
    proj = jax.lax.dot(mean, w_ref[...],
                       precision=jax.lax.Precision.DEFAULT,
                       preferred_element_type=jnp.float32) + b_ref[...]
    out_ref[...] = proj * valid
    valid_ref[...] = jnp.broadcast_to(valid, (B * ROWS, 128))


@functools.partial(jax.jit, static_argnames=())
def _segment_sums(hidden_states, lmask):
    lm3 = lmask.astype(jnp.int32).reshape(B, NCH, 1, CH)
    sums, counts = pl.pallas_call(
        _seg_body,
        grid=(B, NCH),
        in_specs=[
            pl.BlockSpec((1, 1, 1, CH), lambda b, c: (b, c, 0, 0)),
            pl.BlockSpec((1, CH, H), lambda b, c: (b, c, 0)),
        ],
        out_specs=[
            pl.BlockSpec((1, ROWS, H), lambda b, c: (b, 0, 0)),
            pl.BlockSpec((1, ROWS, 128), lambda b, c: (b, 0, 0)),
        ],
        out_shape=[
            jax.ShapeDtypeStruct((B, ROWS, H), jnp.float32),
            jax.ShapeDtypeStruct((B, ROWS, 128), jnp.float32),
        ],
    )(lm3, hidden_states)
    return sums, counts


def _finish(sums, counts, W_label, b_label):
    out, valid = pl.pallas_call(
        _fin_body,
        out_shape=[
            jax.ShapeDtypeStruct((B * ROWS, H), jnp.float32),
            jax.ShapeDtypeStruct((B * ROWS, 128), jnp.float32),
        ],
    )(sums.reshape(B * ROWS, H), counts.reshape(B * ROWS, 128),
      W_label, b_label.reshape(1, H))
    return out, valid


def kernel(hidden_states, lmask, input_ids, attention_mask, W_label, b_label):
    sums, counts = _segment_sums(hidden_states, lmask)
    out, valid = _finish(sums, counts, W_label, b_label)
    out3 = out.reshape(B, ROWS, H)[:, 1:MAX_LABEL + 1, :]
    aggregated = out3.reshape(B * MAX_LABEL, H)
    valid_mask = (valid.reshape(B, ROWS, 128)[:, 1:MAX_LABEL + 1, 0] > 0
                  ).reshape(B * MAX_LABEL)
    all_batch_ids = jnp.repeat(jnp.arange(B), MAX_LABEL)
    all_label_ids = jnp.tile(jnp.arange(1, MAX_LABEL + 1), B)
    return aggregated, all_batch_ids, all_label_ids, valid_mask
